# Initial kernel scaffold; baseline (speedup 1.0000x reference)
import jax
import jax.numpy as jnp
from jax.experimental import pallas as pl


def kernel(h, prev_h, W_pred, b_pred, W_g1, b_g1, W_g2, b_g2, W_w, b_w, w0, state_embed, mu, sigma, slot_state):
    raise NotImplementedError("write your pallas kernel here")


# speedup vs baseline: 1.4745x; 1.4745x over previous
import jax
import jax.numpy as jnp
from jax.experimental import pallas as pl
from jax.experimental.pallas import tpu as pltpu

B = 2048
D = 768
S = 8
GH = 64
SE = 8

TB = 512


def _body(mu_ref, sigma_ref, slot_state_ref, bg2_ref,
          h_ref, ph_ref, Wp_ref, bp_ref, Wg1h_ref, wg1z_ref, Wg1se_ref,
          bg1_ref, Wg2_ref, Ww_ref, bw_ref, w0_ref, se_ref, out_ref):
    s = pl.program_id(0)

    mu_s = mu_ref[s]
    sig_s = jnp.maximum(sigma_ref[s], 1e-3)
    st = slot_state_ref[s]
    gain = jnp.where(st == 0, 1.0, jnp.where(st == 1, 0.5, 0.1))

    h = h_ref[...]
    ph = ph_ref[...]

    pred = jnp.dot(ph, Wp_ref[0], preferred_element_type=jnp.float32)
    pred = pred + bp_ref[0]
    diff = h - pred
    err = (0.5 / D) * jnp.sum(diff * diff, axis=1, keepdims=True)
    z = (err - mu_s) / sig_s

    sel = (jax.lax.broadcasted_iota(jnp.int32, (3, SE), 0) == st)
    se_vec = jnp.sum(jnp.where(sel, se_ref[...], 0.0), axis=0)
    se_term = jnp.sum(se_vec[:, None] * Wg1se_ref[0], axis=0)

    hg = jnp.dot(h, Wg1h_ref[0], preferred_element_type=jnp.float32)
    hg = hg + z * wg1z_ref[0] + se_term[None, :] + bg1_ref[0]
    hg = jnp.maximum(hg, 0.0)
    g = jax.nn.sigmoid(
        jnp.dot(hg, Wg2_ref[0], preferred_element_type=jnp.float32)
        + bg2_ref[s])
    ge = g * gain

    write = jnp.dot(h, Ww_ref[0], preferred_element_type=jnp.float32)
    write = write + bw_ref[0]
    out_ref[...] = (1.0 - ge) * w0_ref[0] + ge * write


def kernel(h, prev_h, W_pred, b_pred, W_g1, b_g1, W_g2, b_g2, W_w, b_w, w0,
           state_embed, mu, sigma, slot_state):
    Wg1h = W_g1[:, :D, :]
    wg1z = W_g1[:, D, :].reshape(S, 1, GH)
    Wg1se = W_g1[:, D + 1:, :]

    smem = pl.BlockSpec(memory_space=pltpu.SMEM)
    nb = B // TB
    grid = (S, nb)

    out = pl.pallas_call(
        _body,
        grid=grid,
        in_specs=[
            smem,
            smem,
            smem,
            smem,
            pl.BlockSpec((TB, D), lambda s, b: (b, 0)),
            pl.BlockSpec((TB, D), lambda s, b: (b, 0)),
            pl.BlockSpec((1, D, D), lambda s, b: (s, 0, 0)),
            pl.BlockSpec((1, 1, D), lambda s, b: (s, 0, 0)),
            pl.BlockSpec((1, D, GH), lambda s, b: (s, 0, 0)),
            pl.BlockSpec((1, 1, GH), lambda s, b: (s, 0, 0)),
            pl.BlockSpec((1, SE, GH), lambda s, b: (s, 0, 0)),
            pl.BlockSpec((1, 1, GH), lambda s, b: (s, 0, 0)),
            pl.BlockSpec((1, GH, 1), lambda s, b: (s, 0, 0)),
            pl.BlockSpec((1, D, D), lambda s, b: (s, 0, 0)),
            pl.BlockSpec((1, 1, D), lambda s, b: (s, 0, 0)),
            pl.BlockSpec((1, 1, D), lambda s, b: (s, 0, 0)),
            pl.BlockSpec((3, SE), lambda s, b: (0, 0)),
        ],
        out_specs=pl.BlockSpec((TB, D), lambda s, b: (b, s)),
        out_shape=jax.ShapeDtypeStruct((B, S * D), jnp.float32),
        compiler_params=pltpu.CompilerParams(
            dimension_semantics=("arbitrary", "arbitrary"),
        ),
    )(
        mu, sigma, slot_state, b_g2.reshape(S),
        h, prev_h, W_pred, b_pred.reshape(S, 1, D), Wg1h, wg1z, Wg1se,
        b_g1.reshape(S, 1, GH), W_g2, W_w, b_w.reshape(S, 1, D),
        w0.reshape(S, 1, D), state_embed,
    )
    return out
